# baseline (speedup 1.0000x reference)
import functools

import jax
import jax.numpy as jnp
from jax import lax
from jax.experimental import pallas as pl
from jax.experimental.pallas import tpu as pltpu
from jax.experimental.pallas import tpu_sc as plsc

NV = 1_000_000
NT = 2_000_000
NVAL2 = 1_002_048 + 512
NSEL = 500_000
NPAD = 1 << 20
NC, NSUB = 2, 16
NW = NC * NSUB
NSEL_PAD = 1 << 19
TRI_PER_TILE = NSEL_PAD // NW
K1_CHUNK = 4096
FL_PER_TILE = NPAD // NW
S2 = 2048
TAIL = 576
VSEG = S2 + 8

_mesh = plsc.VectorSubcoreMesh(
    core_axis_name="c", subcore_axis_name="s", num_cores=NC, num_subcores=NSUB
)

_i32 = jnp.int32
_params = pltpu.CompilerParams(needs_layout_passes=False)


def _wid():
    return lax.axis_index("s") * NC + lax.axis_index("c")


@functools.partial(
    pl.kernel,
    out_type=(
        jax.ShapeDtypeStruct((NPAD,), _i32),
        jax.ShapeDtypeStruct((NPAD,), _i32),
    ),
    mesh=_mesh,
    compiler_params=_params,
    scratch_types=[
        pltpu.VMEM((K1_CHUNK,), _i32),
        pltpu.VMEM((K1_CHUNK,), _i32),
        pltpu.VMEM((K1_CHUNK,), _i32),
        pltpu.VMEM((K1_CHUNK,), _i32),
        pltpu.VMEM((K1_CHUNK,), _i32),
        pltpu.VMEM((K1_CHUNK,), _i32),
        pltpu.VMEM((K1_CHUNK,), _i32),
        pltpu.VMEM((K1_CHUNK,), _i32),
        pltpu.VMEM((K1_CHUNK,), _i32),
        pltpu.VMEM((K1_CHUNK,), _i32),
        pltpu.VMEM((K1_CHUNK,), _i32),
        pltpu.VMEM((K1_CHUNK,), _i32),
        pltpu.VMEM((K1_CHUNK,), _i32),
        pltpu.VMEM((4096,), _i32),
        pltpu.VMEM_SHARED((NPAD,), _i32),
        pltpu.SemaphoreType.DMA,
        pltpu.SemaphoreType.DMA,
        pltpu.SemaphoreType.DMA,
        pltpu.SemaphoreType.DMA,
        pltpu.SemaphoreType.DMA,
        pltpu.SemaphoreType.DMA,
    ],
)
def _k1(tri_p, tri_idx_p, f0, f1, tbA, tbB, i1A, i1B, i2A, i2B,
        g0A, g0B, g1A, g1B, g2A, g2B,
        ones, zbuf, fsp, ga, gb, gc, sa, sb, sc):
    tbufs = (tbA, tbB)
    i1bs = (i1A, i1B)
    i2bs = (i2A, i2B)
    gbufs = ((g0A, g1A, g2A), (g0B, g1B, g2B))
    c = lax.axis_index("c")
    s = lax.axis_index("s")
    wid = s * NC + c
    gsems = (ga, gb, gc)
    ssems = (sa, sb, sc)
    NCHUNK = TRI_PER_TILE // K1_CHUNK

    def fill1(i, _):
        ones[pl.ds(i * 16, 16)] = jnp.full((16,), 1, _i32)
        return 0

    lax.fori_loop(0, K1_CHUNK // 16, fill1, 0)

    def fillz(i, _):
        zbuf[pl.ds(i * 16, 16)] = jnp.zeros((16,), _i32)
        return 0

    lax.fori_loop(0, 256, fillz, 0)

    base = wid * TRI_PER_TILE

    def fetch(k, p):
        tb, i1, i2 = tbufs[p], i1bs[p], i2bs[p]
        pltpu.sync_copy(tri_idx_p.at[pl.ds(base + k * K1_CHUNK, K1_CHUNK)], tb)

        def cvt(i, _):
            t = tb[pl.ds(i * 16, 16)]
            i1[pl.ds(i * 16, 16)] = t + NT
            i2[pl.ds(i * 16, 16)] = t + 2 * NT
            return 0

        lax.fori_loop(0, K1_CHUNK // 16, cvt, 0)
        return (pltpu.async_copy(tri_p.at[tb], gbufs[p][0], gsems[0]),
                pltpu.async_copy(tri_p.at[i1], gbufs[p][1], gsems[1]),
                pltpu.async_copy(tri_p.at[i2], gbufs[p][2], gsems[2]))

    def scat(p):
        return (pltpu.async_copy(ones, fsp.at[gbufs[p][0]], ssems[0]),
                pltpu.async_copy(ones, fsp.at[gbufs[p][1]], ssems[1]),
                pltpu.async_copy(ones, fsp.at[gbufs[p][2]], ssems[2]))

    g_in = fetch(0, 0)
    zslice = NPAD // NSUB

    def z(i, _):
        pltpu.sync_copy(zbuf, fsp.at[pl.ds(s * zslice + i * 4096, 4096)])
        return 0

    lax.fori_loop(0, zslice // 4096, z, 0)
    plsc.subcore_barrier()

    s_in = None
    for k in range(NCHUNK):
        p = k % 2
        for d in g_in:
            d.wait()
        if s_in is not None:
            for d in s_in:
                d.wait()
        s_in = scat(p)
        if k + 1 < NCHUNK:
            g_in = fetch(k + 1, 1 - p)
    for d in s_in:
        d.wait()
    plsc.subcore_barrier()

    @pl.when(c == 0)
    def _():
        pltpu.sync_copy(fsp.at[pl.ds(s * zslice, zslice)],
                        f0.at[pl.ds(s * zslice, zslice)])

    @pl.when(c == 1)
    def _():
        pltpu.sync_copy(fsp.at[pl.ds(s * zslice, zslice)],
                        f1.at[pl.ds(s * zslice, zslice)])


@functools.partial(
    pl.kernel,
    out_type=(
        jax.ShapeDtypeStruct((NW, 16), _i32),
        jax.ShapeDtypeStruct((NPAD,), _i32),
    ),
    mesh=_mesh,
    compiler_params=_params,
    scratch_types=[
        pltpu.VMEM((2048,), _i32),
        pltpu.VMEM((2048,), _i32),
        pltpu.VMEM((2048,), _i32),
        pltpu.VMEM((16,), _i32),
    ],
)
def _k2a(f0, f1, sums, flor, fa, fb, fo, sb):
    wid = _wid()
    base = wid * FL_PER_TILE

    def blk(i, accv):
        pltpu.sync_copy(f0.at[pl.ds(base + i * 2048, 2048)], fa)
        pltpu.sync_copy(f1.at[pl.ds(base + i * 2048, 2048)], fb)

        def inner(j, a):
            v = fa[pl.ds(j * 16, 16)] | fb[pl.ds(j * 16, 16)]
            fo[pl.ds(j * 16, 16)] = v
            return a + v

        accv = lax.fori_loop(0, 128, inner, accv)
        pltpu.sync_copy(fo, flor.at[pl.ds(base + i * 2048, 2048)])
        return accv

    accv = lax.fori_loop(0, FL_PER_TILE // 2048, blk, jnp.zeros((16,), _i32))
    tot = jnp.sum(accv)
    sb[pl.ds(0, 16)] = jnp.broadcast_to(tot, (16,))
    pltpu.sync_copy(sb, sums.at[wid])


@functools.partial(
    pl.kernel,
    out_type=jax.ShapeDtypeStruct((3 * NV,), jnp.float32),
    mesh=_mesh,
    compiler_params=_params,
    scratch_types=[
        pltpu.VMEM((NW, 16), _i32),
        pltpu.VMEM((S2,), _i32),
        pltpu.VMEM((3 * VSEG,), jnp.float32),
        pltpu.VMEM((3 * S2,), jnp.float32),
        pltpu.SemaphoreType.DMA,
        pltpu.SemaphoreType.DMA,
        pltpu.SemaphoreType.DMA,
    ],
)
def _k2c(flor, sums, val_p, out_p,
         svmem, fla, vbuf, outb, sa, sb_, sc_):
    wid = _wid()
    pltpu.sync_copy(sums, svmem)
    iota = lax.iota(_i32, 16)
    zeros16 = jnp.zeros((16,), _i32)
    clo = plsc.load_gather(svmem, [iota, zeros16])
    chi = plsc.load_gather(svmem, [iota + 16, zeros16])
    off = (jnp.sum(jnp.where(iota < jnp.minimum(wid, 16), clo, 0))
           + jnp.sum(jnp.where(iota < wid - 16, chi, 0)))

    base_v = wid * FL_PER_TILE
    sems = (sa, sb_, sc_)

    def do_sub(v0, cum, S):
        pltpu.sync_copy(flor.at[pl.ds(v0, S)], fla.at[pl.ds(0, S)])
        st8 = pl.multiple_of(cum & jnp.int32(-8), 8)
        sh = cum - st8
        dw = [pltpu.async_copy(val_p.at[pl.ds(m * NVAL2 + st8, S + 8)],
                               vbuf.at[pl.ds(m * VSEG, S + 8)], sems[m])
              for m in range(3)]
        for d in dw:
            d.wait()

        def opass(i, lb):
            f = fla[pl.ds(i * 16, 16)]
            msk = f > 0
            for m in range(3):
                g = plsc.load_expanded(vbuf.at[pl.ds(m * VSEG + lb, 16)], mask=msk)
                outb[pl.ds(m * S2 + i * 16, 16)] = jnp.where(msk, g, 0.0)
            return lb + jnp.sum(f)

        lb_end = lax.fori_loop(0, S // 16, opass, sh)
        do = [pltpu.async_copy(outb.at[pl.ds(m * S2, S)],
                               out_p.at[pl.ds(m * NV + v0, S)], sems[m])
              for m in range(3)]
        for d in do:
            d.wait()
        return cum + (lb_end - sh)

    nfull = jnp.clip((NV - base_v) // S2, 0, FL_PER_TILE // S2)

    def mainb(k, cum):
        return do_sub(base_v + k * S2, cum, S2)

    cum = lax.fori_loop(0, nfull, mainb, off)

    @pl.when((nfull < FL_PER_TILE // S2) & (base_v + nfull * S2 < NV))
    def _():
        do_sub(base_v + nfull * S2, cum, TAIL)


def kernel(vertices, triangles, tri_idx, values):
    tri_p = triangles.T.reshape(-1)
    pad = jnp.broadcast_to(tri_idx[:1], (NSEL_PAD - NSEL,))
    tri_idx_p = jnp.concatenate([tri_idx, pad])
    vals_in, _ = lax.optimization_barrier((values[:NVAL2], tri_p))
    val_p = vals_in.T.reshape(-1)
    f0, f1 = _k1(tri_p, tri_idx_p)
    sums, flor = _k2a(f0, f1)
    delta_p = _k2c(flor, sums, val_p)
    return vertices + delta_p.reshape(3, NV).T


# speedup vs baseline: 1.7811x; 1.7811x over previous
import functools

import jax
import jax.numpy as jnp
from jax import lax
from jax.experimental import pallas as pl
from jax.experimental.pallas import tpu as pltpu
from jax.experimental.pallas import tpu_sc as plsc

NV = 1_000_000
NT = 2_000_000
NVAL2 = 1_002_048 + 512
NSEL = 500_000
NPAD = 1 << 20
NC, NSUB = 2, 16
NW = NC * NSUB
NSEL_PAD = 1 << 19
TRI_PER_TILE = NSEL_PAD // NW
K1_CHUNK = 4096
FL_PER_TILE = NPAD // NW
S2 = 2048
TAIL = 576
VSEG = S2 + 8

_mesh = plsc.VectorSubcoreMesh(
    core_axis_name="c", subcore_axis_name="s", num_cores=NC, num_subcores=NSUB
)

_i32 = jnp.int32
_params = pltpu.CompilerParams(needs_layout_passes=False)


def _wid():
    return lax.axis_index("s") * NC + lax.axis_index("c")


@functools.partial(
    pl.kernel,
    out_type=(
        jax.ShapeDtypeStruct((NPAD,), _i32),
        jax.ShapeDtypeStruct((NPAD,), _i32),
    ),
    mesh=_mesh,
    compiler_params=_params,
    scratch_types=[
        pltpu.VMEM((K1_CHUNK,), _i32),
        pltpu.VMEM((K1_CHUNK,), _i32),
        pltpu.VMEM((K1_CHUNK,), _i32),
        pltpu.VMEM((K1_CHUNK,), _i32),
        pltpu.VMEM((K1_CHUNK,), _i32),
        pltpu.VMEM((K1_CHUNK,), _i32),
        pltpu.VMEM((K1_CHUNK,), _i32),
        pltpu.VMEM((K1_CHUNK,), _i32),
        pltpu.VMEM((K1_CHUNK,), _i32),
        pltpu.VMEM((4096,), _i32),
        pltpu.VMEM_SHARED((NPAD,), _i32),
        pltpu.SemaphoreType.DMA,
        pltpu.SemaphoreType.DMA,
        pltpu.SemaphoreType.DMA,
        pltpu.SemaphoreType.DMA,
        pltpu.SemaphoreType.DMA,
        pltpu.SemaphoreType.DMA,
    ],
)
def _k1(tri0, tri1, tri2, tri_idx_p, f0, f1, tbA, tbB,
        g0A, g0B, g1A, g1B, g2A, g2B,
        ones, zbuf, fsp, ga, gb, gc, sa, sb, sc):
    tbufs = (tbA, tbB)
    tris = (tri0, tri1, tri2)
    gbufs = ((g0A, g1A, g2A), (g0B, g1B, g2B))
    c = lax.axis_index("c")
    s = lax.axis_index("s")
    wid = s * NC + c
    gsems = (ga, gb, gc)
    ssems = (sa, sb, sc)
    NCHUNK = TRI_PER_TILE // K1_CHUNK

    def fill1(i, _):
        ones[pl.ds(i * 16, 16)] = jnp.full((16,), 1, _i32)
        return 0

    lax.fori_loop(0, K1_CHUNK // 16, fill1, 0)

    def fillz(i, _):
        zbuf[pl.ds(i * 16, 16)] = jnp.zeros((16,), _i32)
        return 0

    lax.fori_loop(0, 256, fillz, 0)

    base = wid * TRI_PER_TILE

    def fetch(k, p):
        tb = tbufs[p]
        pltpu.sync_copy(tri_idx_p.at[pl.ds(base + k * K1_CHUNK, K1_CHUNK)], tb)
        return tuple(pltpu.async_copy(tris[m].at[tb], gbufs[p][m], gsems[m])
                     for m in range(3))

    def scat(p):
        return (pltpu.async_copy(ones, fsp.at[gbufs[p][0]], ssems[0]),
                pltpu.async_copy(ones, fsp.at[gbufs[p][1]], ssems[1]),
                pltpu.async_copy(ones, fsp.at[gbufs[p][2]], ssems[2]))

    g_in = fetch(0, 0)
    zslice = NPAD // NSUB

    def z(i, _):
        pltpu.sync_copy(zbuf, fsp.at[pl.ds(s * zslice + i * 4096, 4096)])
        return 0

    lax.fori_loop(0, zslice // 4096, z, 0)
    plsc.subcore_barrier()

    s_in = None
    for k in range(NCHUNK):
        p = k % 2
        for d in g_in:
            d.wait()
        if s_in is not None:
            for d in s_in:
                d.wait()
        s_in = scat(p)
        if k + 1 < NCHUNK:
            g_in = fetch(k + 1, 1 - p)
    for d in s_in:
        d.wait()
    plsc.subcore_barrier()

    @pl.when(c == 0)
    def _():
        pltpu.sync_copy(fsp.at[pl.ds(s * zslice, zslice)],
                        f0.at[pl.ds(s * zslice, zslice)])

    @pl.when(c == 1)
    def _():
        pltpu.sync_copy(fsp.at[pl.ds(s * zslice, zslice)],
                        f1.at[pl.ds(s * zslice, zslice)])


@functools.partial(
    pl.kernel,
    out_type=(
        jax.ShapeDtypeStruct((NW, 16), _i32),
        jax.ShapeDtypeStruct((NPAD,), _i32),
    ),
    mesh=_mesh,
    compiler_params=_params,
    scratch_types=[
        pltpu.VMEM((2048,), _i32),
        pltpu.VMEM((2048,), _i32),
        pltpu.VMEM((2048,), _i32),
        pltpu.VMEM((16,), _i32),
    ],
)
def _k2a(f0, f1, sums, flor, fa, fb, fo, sb):
    wid = _wid()
    base = wid * FL_PER_TILE

    def blk(i, accv):
        pltpu.sync_copy(f0.at[pl.ds(base + i * 2048, 2048)], fa)
        pltpu.sync_copy(f1.at[pl.ds(base + i * 2048, 2048)], fb)

        def inner(j, a):
            v = fa[pl.ds(j * 16, 16)] | fb[pl.ds(j * 16, 16)]
            fo[pl.ds(j * 16, 16)] = v
            return a + v

        accv = lax.fori_loop(0, 128, inner, accv)
        pltpu.sync_copy(fo, flor.at[pl.ds(base + i * 2048, 2048)])
        return accv

    accv = lax.fori_loop(0, FL_PER_TILE // 2048, blk, jnp.zeros((16,), _i32))
    tot = jnp.sum(accv)
    sb[pl.ds(0, 16)] = jnp.broadcast_to(tot, (16,))
    pltpu.sync_copy(sb, sums.at[wid])


@functools.partial(
    pl.kernel,
    out_type=jax.ShapeDtypeStruct((3 * NV,), jnp.float32),
    mesh=_mesh,
    compiler_params=_params,
    scratch_types=[
        pltpu.VMEM((NW, 16), _i32),
        pltpu.VMEM((S2,), _i32),
        pltpu.VMEM((3 * VSEG,), jnp.float32),
        pltpu.VMEM((3 * S2,), jnp.float32),
        pltpu.SemaphoreType.DMA,
        pltpu.SemaphoreType.DMA,
        pltpu.SemaphoreType.DMA,
    ],
)
def _k2c(flor, sums, val0, val1, val2, out_p,
         svmem, fla, vbuf, outb, sa, sb_, sc_):
    vals = (val0, val1, val2)
    wid = _wid()
    pltpu.sync_copy(sums, svmem)
    iota = lax.iota(_i32, 16)
    zeros16 = jnp.zeros((16,), _i32)
    clo = plsc.load_gather(svmem, [iota, zeros16])
    chi = plsc.load_gather(svmem, [iota + 16, zeros16])
    off = (jnp.sum(jnp.where(iota < jnp.minimum(wid, 16), clo, 0))
           + jnp.sum(jnp.where(iota < wid - 16, chi, 0)))

    base_v = wid * FL_PER_TILE
    sems = (sa, sb_, sc_)

    def do_sub(v0, cum, S):
        pltpu.sync_copy(flor.at[pl.ds(v0, S)], fla.at[pl.ds(0, S)])
        st8 = pl.multiple_of(cum & jnp.int32(-8), 8)
        sh = cum - st8
        dw = [pltpu.async_copy(vals[m].at[pl.ds(st8, S + 8)],
                               vbuf.at[pl.ds(m * VSEG, S + 8)], sems[m])
              for m in range(3)]
        for d in dw:
            d.wait()

        def opass(i, lb):
            f = fla[pl.ds(i * 16, 16)]
            msk = f > 0
            for m in range(3):
                g = plsc.load_expanded(vbuf.at[pl.ds(m * VSEG + lb, 16)], mask=msk)
                outb[pl.ds(m * S2 + i * 16, 16)] = jnp.where(msk, g, 0.0)
            return lb + jnp.sum(f)

        lb_end = lax.fori_loop(0, S // 16, opass, sh)
        do = [pltpu.async_copy(outb.at[pl.ds(m * S2, S)],
                               out_p.at[pl.ds(m * NV + v0, S)], sems[m])
              for m in range(3)]
        for d in do:
            d.wait()
        return cum + (lb_end - sh)

    nfull = jnp.clip((NV - base_v) // S2, 0, FL_PER_TILE // S2)

    def mainb(k, cum):
        return do_sub(base_v + k * S2, cum, S2)

    cum = lax.fori_loop(0, nfull, mainb, off)

    @pl.when((nfull < FL_PER_TILE // S2) & (base_v + nfull * S2 < NV))
    def _():
        do_sub(base_v + nfull * S2, cum, TAIL)


def kernel(vertices, triangles, tri_idx, values):
    tri0, tri1, tri2 = (triangles[:, m] for m in range(3))
    val0, val1, val2 = (values[:NVAL2, m] for m in range(3))
    pad = jnp.broadcast_to(tri_idx[:1], (NSEL_PAD - NSEL,))
    tri_idx_p = jnp.concatenate([tri_idx, pad])
    f0, f1 = _k1(tri0, tri1, tri2, tri_idx_p)
    sums, flor = _k2a(f0, f1)
    delta_p = _k2c(flor, sums, val0, val1, val2)
    return vertices + delta_p.reshape(3, NV).T


# speedup vs baseline: 2.3135x; 1.2989x over previous
import functools

import jax
import jax.numpy as jnp
from jax import lax
from jax.experimental import pallas as pl
from jax.experimental.pallas import tpu as pltpu
from jax.experimental.pallas import tpu_sc as plsc

NV = 1_000_000
NT = 2_000_000
NVAL2 = 1_002_048 + 512
NSEL = 500_000
NPAD = 1 << 20
NC, NSUB = 2, 16
NW = NC * NSUB
NSEL_PAD = 500_736
TRI_PER_TILE = NSEL_PAD // NW
K1_CHUNK = TRI_PER_TILE // 3
FL_PER_TILE = NPAD // NW
S2 = 2048
TAIL = 576
VSEG = S2 + 8

_mesh = plsc.VectorSubcoreMesh(
    core_axis_name="c", subcore_axis_name="s", num_cores=NC, num_subcores=NSUB
)

_i32 = jnp.int32
_params = pltpu.CompilerParams(needs_layout_passes=False)


def _wid():
    return lax.axis_index("s") * NC + lax.axis_index("c")


@functools.partial(
    pl.kernel,
    out_type=(
        jax.ShapeDtypeStruct((NPAD,), _i32),
        jax.ShapeDtypeStruct((NPAD,), _i32),
    ),
    mesh=_mesh,
    compiler_params=_params,
    scratch_types=[
        pltpu.VMEM((K1_CHUNK,), _i32),
        pltpu.VMEM((K1_CHUNK,), _i32),
        pltpu.VMEM((K1_CHUNK,), _i32),
        pltpu.VMEM((K1_CHUNK,), _i32),
        pltpu.VMEM((K1_CHUNK,), _i32),
        pltpu.VMEM((K1_CHUNK,), _i32),
        pltpu.VMEM((K1_CHUNK,), _i32),
        pltpu.VMEM((K1_CHUNK,), _i32),
        pltpu.VMEM((K1_CHUNK,), _i32),
        pltpu.VMEM((4096,), _i32),
        pltpu.VMEM_SHARED((NPAD,), _i32),
        pltpu.SemaphoreType.DMA,
        pltpu.SemaphoreType.DMA,
        pltpu.SemaphoreType.DMA,
        pltpu.SemaphoreType.DMA,
        pltpu.SemaphoreType.DMA,
        pltpu.SemaphoreType.DMA,
    ],
)
def _k1(tri0, tri1, tri2, tri_idx_p, f0, f1, tbA, tbB,
        g0A, g0B, g1A, g1B, g2A, g2B,
        ones, zbuf, fsp, ga, gb, gc, sa, sb, sc):
    tbufs = (tbA, tbB)
    tris = (tri0, tri1, tri2)
    gbufs = ((g0A, g1A, g2A), (g0B, g1B, g2B))
    c = lax.axis_index("c")
    s = lax.axis_index("s")
    wid = s * NC + c
    gsems = (ga, gb, gc)
    ssems = (sa, sb, sc)
    NCHUNK = TRI_PER_TILE // K1_CHUNK

    def fill1(i, _):
        ones[pl.ds(i * 16, 16)] = jnp.full((16,), 1, _i32)
        return 0

    lax.fori_loop(0, K1_CHUNK // 16, fill1, 0)

    def fillz(i, _):
        zbuf[pl.ds(i * 16, 16)] = jnp.zeros((16,), _i32)
        return 0

    lax.fori_loop(0, 256, fillz, 0)

    base = wid * TRI_PER_TILE

    def fetch(k, p):
        tb = tbufs[p]
        pltpu.sync_copy(tri_idx_p.at[pl.ds(base + k * K1_CHUNK, K1_CHUNK)], tb)
        return tuple(pltpu.async_copy(tris[m].at[tb], gbufs[p][m], gsems[m])
                     for m in range(3))

    def scat(p):
        return (pltpu.async_copy(ones, fsp.at[gbufs[p][0]], ssems[0]),
                pltpu.async_copy(ones, fsp.at[gbufs[p][1]], ssems[1]),
                pltpu.async_copy(ones, fsp.at[gbufs[p][2]], ssems[2]))

    g_in = fetch(0, 0)
    zslice = NPAD // NSUB

    def z(i, _):
        pltpu.sync_copy(zbuf, fsp.at[pl.ds(s * zslice + i * 4096, 4096)])
        return 0

    lax.fori_loop(0, zslice // 4096, z, 0)
    plsc.subcore_barrier()

    s_in = None
    for k in range(NCHUNK):
        p = k % 2
        for d in g_in:
            d.wait()
        if s_in is not None:
            for d in s_in:
                d.wait()
        s_in = scat(p)
        if k + 1 < NCHUNK:
            g_in = fetch(k + 1, 1 - p)
    for d in s_in:
        d.wait()
    plsc.subcore_barrier()

    @pl.when(c == 0)
    def _():
        pltpu.sync_copy(fsp.at[pl.ds(s * zslice, zslice)],
                        f0.at[pl.ds(s * zslice, zslice)])

    @pl.when(c == 1)
    def _():
        pltpu.sync_copy(fsp.at[pl.ds(s * zslice, zslice)],
                        f1.at[pl.ds(s * zslice, zslice)])


@functools.partial(
    pl.kernel,
    out_type=(
        jax.ShapeDtypeStruct((NW, 16), _i32),
        jax.ShapeDtypeStruct((NPAD,), _i32),
        jax.ShapeDtypeStruct((NW, 64), _i32),
    ),
    mesh=_mesh,
    compiler_params=_params,
    scratch_types=[
        pltpu.VMEM((2048,), _i32),
        pltpu.VMEM((2048,), _i32),
        pltpu.VMEM((2048,), _i32),
        pltpu.VMEM((64,), _i32),
        pltpu.VMEM((16,), _i32),
    ],
)
def _k2a(f0, f1, sums, flor, gcnt, fa, fb, fo, cb, sb):
    wid = _wid()
    base = wid * FL_PER_TILE

    iota16 = lax.iota(_i32, 16)

    def blk(i, carry):
        accv, cvec = carry
        pltpu.sync_copy(f0.at[pl.ds(base + i * 2048, 2048)], fa)
        pltpu.sync_copy(f1.at[pl.ds(base + i * 2048, 2048)], fb)
        for q in range(4):
            def inner(j, a):
                v = fa[pl.ds(q * 512 + j * 16, 16)] | fb[pl.ds(q * 512 + j * 16, 16)]
                fo[pl.ds(q * 512 + j * 16, 16)] = v
                return a + v

            acc2 = lax.fori_loop(0, 32, inner, jnp.zeros((16,), _i32))
            cnt = jnp.sum(acc2)
            lane = (i * 4 + q) % 16
            cvec = cvec + jnp.where(iota16 == lane, cnt, 0)
            accv = accv + acc2
        pltpu.sync_copy(fo, flor.at[pl.ds(base + i * 2048, 2048)])
        flushed = (i % 4) == 3

        @pl.when(flushed)
        def _():
            cb[pl.ds((i // 4) * 16, 16)] = cvec

        cvec = jnp.where(flushed, jnp.zeros((16,), _i32), cvec)
        return accv, cvec

    accv, _unused = lax.fori_loop(
        0, FL_PER_TILE // 2048, blk,
        (jnp.zeros((16,), _i32), jnp.zeros((16,), _i32)))
    tot = jnp.sum(accv)
    sb[pl.ds(0, 16)] = jnp.broadcast_to(tot, (16,))
    pltpu.sync_copy(sb, sums.at[wid])
    pltpu.sync_copy(cb, gcnt.at[wid])


@functools.partial(
    pl.kernel,
    out_type=jax.ShapeDtypeStruct((3 * NV,), jnp.float32),
    mesh=_mesh,
    compiler_params=_params,
    scratch_types=[
        pltpu.VMEM((NW, 16), _i32),
        pltpu.VMEM((64,), _i32),
        pltpu.VMEM((80,), _i32),
        pltpu.VMEM((S2,), _i32),
        pltpu.VMEM((3 * VSEG,), jnp.float32),
        pltpu.VMEM((3 * S2,), jnp.float32),
        pltpu.SemaphoreType.DMA,
        pltpu.SemaphoreType.DMA,
        pltpu.SemaphoreType.DMA,
    ],
)
def _k2c(flor, sums, gcnt, val0, val1, val2, out_p,
         svmem, gcv, pb, fla, vbuf, outb, sa, sb_, sc_):
    vals = (val0, val1, val2)
    wid = _wid()
    pltpu.sync_copy(sums, svmem)
    pltpu.sync_copy(gcnt.at[wid], gcv)
    iota = lax.iota(_i32, 16)
    zeros16 = jnp.zeros((16,), _i32)
    clo = plsc.load_gather(svmem, [iota, zeros16])
    chi = plsc.load_gather(svmem, [iota + 16, zeros16])
    off = (jnp.sum(jnp.where(iota < jnp.minimum(wid, 16), clo, 0))
           + jnp.sum(jnp.where(iota < wid - 16, chi, 0)))

    carry = off
    for q in range(4):
        v = gcv[pl.ds(q * 16, 16)]
        inc = plsc.cumsum(v)
        pb[pl.ds(q * 16, 16)] = (inc - v) + carry
        carry = carry + jnp.sum(v)

    base_v = wid * FL_PER_TILE
    sems = (sa, sb_, sc_)

    def do_sub(k, groups):
        S = sum(groups)
        v0 = base_v + k * S2
        pbv = pb[pl.ds(k * 4, 16)]
        cum = pbv[0]
        pltpu.sync_copy(flor.at[pl.ds(v0, S)], fla.at[pl.ds(0, S)])
        st8 = pl.multiple_of(cum & jnp.int32(-8), 8)
        dw = [pltpu.async_copy(vals[m].at[pl.ds(st8, S + 8)],
                               vbuf.at[pl.ds(m * VSEG, S + 8)], sems[m])
              for m in range(3)]
        for d in dw:
            d.wait()

        def opass(i, lbs):
            nlbs = []
            for q, gs in enumerate(groups):
                lb = lbs[q]
                f = fla[pl.ds(q * 512 + i * 16, 16)]
                msk = f > 0
                for m in range(3):
                    g = plsc.load_expanded(vbuf.at[pl.ds(m * VSEG + lb, 16)],
                                           mask=msk)
                    outb[pl.ds(m * S2 + q * 512 + i * 16, 16)] = (
                        jnp.where(msk, g, 0.0))
                nlbs.append(lb + jnp.sum(f))
            return tuple(nlbs)

        lbs0 = tuple(pbv[q] - st8 for q in range(len(groups)))
        niter = max(gs for gs in groups) // 16
        if len(set(groups)) == 1:
            lax.fori_loop(0, niter, opass, lbs0)
        else:
            for q, gs in enumerate(groups):
                def one(i, lb, q=q):
                    f = fla[pl.ds(q * 512 + i * 16, 16)]
                    msk = f > 0
                    for m in range(3):
                        g = plsc.load_expanded(
                            vbuf.at[pl.ds(m * VSEG + lb, 16)], mask=msk)
                        outb[pl.ds(m * S2 + q * 512 + i * 16, 16)] = (
                            jnp.where(msk, g, 0.0))
                    return lb + jnp.sum(f)

                lax.fori_loop(0, gs // 16, one, lbs0[q])
        do = [pltpu.async_copy(outb.at[pl.ds(m * S2, S)],
                               out_p.at[pl.ds(3 * 0 + m * NV + v0, S)], sems[m])
              for m in range(3)]
        for d in do:
            d.wait()

    nfull = jnp.clip((NV - base_v) // S2, 0, FL_PER_TILE // S2)

    def mainb(k, _):
        do_sub(k, [512, 512, 512, 512])
        return 0

    lax.fori_loop(0, nfull, mainb, 0)

    @pl.when((nfull < FL_PER_TILE // S2) & (base_v + nfull * S2 < NV))
    def _():
        do_sub(nfull, [512, 64])


def kernel(vertices, triangles, tri_idx, values):
    tri0, tri1, tri2 = (triangles[:, m] for m in range(3))
    val0, val1, val2 = (values[:NVAL2, m] for m in range(3))
    pad = jnp.broadcast_to(tri_idx[:1], (NSEL_PAD - NSEL,))
    tri_idx_p = jnp.concatenate([tri_idx, pad])
    f0, f1 = _k1(tri0, tri1, tri2, tri_idx_p)
    sums, flor, gcnt = _k2a(f0, f1)
    delta_p = _k2c(flor, sums, gcnt, val0, val1, val2)
    return vertices + delta_p.reshape(3, NV).T
